# Initial kernel scaffold; baseline (speedup 1.0000x reference)
import jax
import jax.numpy as jnp
from jax.experimental import pallas as pl


def kernel(x, edge_index, W_est, b_est, W_est_fc, b_est_fc, W_gnn, b_gnn, W_cls, b_cls):
    raise NotImplementedError("write your pallas kernel here")


# speedup vs baseline: 25.5549x; 25.5549x over previous
import functools

import jax
import jax.numpy as jnp
from jax import lax
from jax.experimental import pallas as pl
from jax.experimental.pallas import tpu as pltpu
from jax.experimental.pallas import tpu_sc as plsc

_HIGH = jax.lax.Precision.HIGHEST


def _u_body(x_ref, we_ref, wef_ref, wg_ref, wc_ref, u_ref):
    w_s = jnp.dot(we_ref[...], wef_ref[...], precision=_HIGH)
    w_y = jnp.dot(wg_ref[...], wc_ref[...], precision=_HIGH)
    w2 = jnp.concatenate([w_s, w_y], axis=1)
    u_ref[...] = jax.lax.dot_general(
        w2, x_ref[...], (((0,), (1,)), ((), ())), precision=_HIGH)


def _fin_body(agg_ref, nd_ref, be_ref, wef_ref, bef_ref, bg_ref, wc_ref,
              bc_ref, out_ref):
    bias_s = jnp.sum(be_ref[0, :] * wef_ref[:, 0]) + bef_ref[0, 0]
    bias_y = jnp.sum(bg_ref[0, :] * wc_ref[:, 0]) + bc_ref[0, 0]
    nd = nd_ref[...]
    o_s = (agg_ref[0, 0] + agg_ref[1, 0]) * nd + bias_s
    o_y = (agg_ref[0, 1] + agg_ref[1, 1]) * nd + bias_y
    out_ref[...] = jnp.stack([o_s, o_y])


def _rsqrt16(x):
    i = lax.bitcast_convert_type(x, jnp.int32)
    y = lax.bitcast_convert_type(jnp.int32(0x5F3759DF) - (i >> 1), jnp.float32)
    for _ in range(3):
        y = y * (1.5 - 0.5 * x * y * y)
    return jnp.where(x > 0.5, y, 0.0)


def _make_sc_kernel(NP, EP):
    R = NP // 16
    G2 = R // 16
    P1 = EP // (16 * 2048)
    P3 = EP // (32 * 2048)
    mesh = plsc.VectorSubcoreMesh(core_axis_name="c", subcore_axis_name="s")

    @functools.partial(
        pl.kernel,
        out_type=[
            jax.ShapeDtypeStruct((2, 2, NP), jnp.float32),
            jax.ShapeDtypeStruct((NP,), jnp.float32),
        ],
        mesh=mesh,
        scratch_types=[
            pltpu.VMEM((16, 128), jnp.int32),
            pltpu.VMEM((16, 128), jnp.int32),
            pltpu.VMEM((128,), jnp.float32),
            pltpu.VMEM((128,), jnp.float32),
            pltpu.VMEM((128,), jnp.float32),
            pltpu.VMEM((R,), jnp.float32),
            pltpu.VMEM((R,), jnp.float32),
            pltpu.VMEM((R,), jnp.float32),
            pltpu.VMEM((R,), jnp.float32),
            pltpu.VMEM((R,), jnp.float32),
            pltpu.VMEM((R,), jnp.float32),
            pltpu.VMEM((R,), jnp.float32),
            pltpu.VMEM_SHARED((NP,), jnp.float32),
            pltpu.VMEM_SHARED((NP,), jnp.float32),
            pltpu.VMEM_SHARED((NP,), jnp.float32),
            pltpu.VMEM_SHARED((NP,), jnp.float32),
            pltpu.VMEM_SHARED((NP,), jnp.float32),
            pltpu.VMEM_SHARED((NP,), jnp.float32),
        ],
    )
    def sc_kernel(u0_hbm, u1_hbm, src_hbm, dst_hbm, ones_hbm, zeros_hbm,
                  agg_out, nd_out,
                  sidx, didx, onesv, msg0, msg1,
                  dov, div, u0v, u1v, us0v, us1v, ndv,
                  dego_s, degi_s, us0_s, us1_s, agg0_s, agg1_s):
        c = lax.axis_index("c")
        s = lax.axis_index("s")
        rows = pl.ds(s * R, R)

        pltpu.sync_copy(ones_hbm, onesv)
        pltpu.sync_copy(zeros_hbm.at[rows], dego_s.at[rows])
        pltpu.sync_copy(zeros_hbm.at[rows], degi_s.at[rows])
        pltpu.sync_copy(zeros_hbm.at[rows], agg0_s.at[rows])
        pltpu.sync_copy(zeros_hbm.at[rows], agg1_s.at[rows])
        plsc.subcore_barrier()

        def deg_blk(b, carry):
            base = pl.ds((s * P1 + b) * 16, 16)
            pltpu.sync_copy(src_hbm.at[base], sidx)
            pltpu.sync_copy(dst_hbm.at[base], didx)
            for j in range(16):
                pltpu.sync_copy(onesv, dego_s.at[sidx.at[j]], add=True)
                pltpu.sync_copy(onesv, degi_s.at[didx.at[j]], add=True)
            return carry
        lax.fori_loop(0, P1, deg_blk, 0)
        plsc.subcore_barrier()

        pltpu.sync_copy(dego_s.at[rows], dov)
        pltpu.sync_copy(degi_s.at[rows], div)
        pltpu.sync_copy(u0_hbm.at[rows], u0v)
        pltpu.sync_copy(u1_hbm.at[rows], u1v)

        def norm_g(g, carry):
            sl = pl.ds(g * 16, 16)
            nsrc = _rsqrt16(dov[sl])
            ndv[sl] = _rsqrt16(div[sl])
            us0v[sl] = u0v[sl] * nsrc
            us1v[sl] = u1v[sl] * nsrc
            return carry
        lax.fori_loop(0, G2, norm_g, 0)
        pltpu.sync_copy(us0v, us0_s.at[rows])
        pltpu.sync_copy(us1v, us1_s.at[rows])

        @pl.when(c == 0)
        def _():
            pltpu.sync_copy(ndv, nd_out.at[rows])
        plsc.subcore_barrier()

        wid = s * 2 + c
        def msg_blk(b, carry):
            base = pl.ds((wid * P3 + b) * 16, 16)
            pltpu.sync_copy(src_hbm.at[base], sidx)
            pltpu.sync_copy(dst_hbm.at[base], didx)
            for j in range(16):
                pltpu.sync_copy(us0_s.at[sidx.at[j]], msg0)
                pltpu.sync_copy(us1_s.at[sidx.at[j]], msg1)
                pltpu.sync_copy(msg0, agg0_s.at[didx.at[j]], add=True)
                pltpu.sync_copy(msg1, agg1_s.at[didx.at[j]], add=True)
            return carry
        lax.fori_loop(0, P3, msg_blk, 0)
        plsc.subcore_barrier()

        pltpu.sync_copy(agg0_s.at[rows], agg_out.at[c, 0, rows])
        pltpu.sync_copy(agg1_s.at[rows], agg_out.at[c, 1, rows])

    return sc_kernel


def kernel(x, edge_index, W_est, b_est, W_est_fc, b_est_fc, W_gnn, b_gnn,
           W_cls, b_cls):
    N, D = x.shape
    E = edge_index.shape[1]

    NP = 256 * ((N + 255) // 256)
    EP = 65536 * ((E + 65535) // 65536)
    if EP > E and NP == N:
        NP += 256

    xp = jnp.zeros((NP, D), x.dtype).at[:N].set(x)

    npad = EP - E
    if npad:
        pad = N + (jnp.arange(npad, dtype=jnp.int32) % (NP - N))
        src = jnp.concatenate([edge_index[0], pad])
        dst = jnp.concatenate([edge_index[1], pad])
    else:
        src = edge_index[0]
        dst = edge_index[1]
    src2d = src.reshape(EP // 128, 128)
    dst2d = dst.reshape(EP // 128, 128)

    u = pl.pallas_call(
        _u_body,
        out_shape=jax.ShapeDtypeStruct((2, NP), jnp.float32),
    )(xp, W_est, W_est_fc, W_gnn, W_cls)

    ones = jnp.ones((128,), jnp.float32)
    zeros = jnp.zeros((NP,), jnp.float32)
    agg, nd = _make_sc_kernel(NP, EP)(u[0], u[1], src2d, dst2d, ones, zeros)

    out2 = pl.pallas_call(
        _fin_body,
        out_shape=jax.ShapeDtypeStruct((2, NP), jnp.float32),
    )(agg, nd, b_est.reshape(1, -1), W_est_fc, b_est_fc.reshape(1, 1),
      b_gnn.reshape(1, -1), W_cls, b_cls.reshape(1, 1))

    return (out2[1, :N, None], out2[0, :N, None])


# speedup vs baseline: 42.5035x; 1.6632x over previous
import functools

import jax
import jax.numpy as jnp
from jax import lax
from jax.experimental import pallas as pl
from jax.experimental.pallas import tpu as pltpu
from jax.experimental.pallas import tpu_sc as plsc

_HIGH = jax.lax.Precision.HIGHEST


def _u_body(x_ref, we_ref, wef_ref, wg_ref, wc_ref, u_ref):
    w_s = jnp.dot(we_ref[...], wef_ref[...], precision=_HIGH)
    w_y = jnp.dot(wg_ref[...], wc_ref[...], precision=_HIGH)
    w2 = jnp.concatenate([w_s, w_y], axis=1)
    u_ref[...] = jax.lax.dot_general(
        w2, x_ref[...], (((0,), (1,)), ((), ())), precision=_HIGH)


def _fin_body(agg_ref, nd_ref, be_ref, wef_ref, bef_ref, bg_ref, wc_ref,
              bc_ref, out_ref):
    bias_s = jnp.sum(be_ref[0, :] * wef_ref[:, 0]) + bef_ref[0, 0]
    bias_y = jnp.sum(bg_ref[0, :] * wc_ref[:, 0]) + bc_ref[0, 0]
    nd = nd_ref[...]
    o_s = (agg_ref[0, 0] + agg_ref[1, 0]) * nd + bias_s
    o_y = (agg_ref[0, 1] + agg_ref[1, 1]) * nd + bias_y
    out_ref[...] = jnp.stack([o_s, o_y])


def _rsqrt16(x):
    i = lax.bitcast_convert_type(x, jnp.int32)
    y = lax.bitcast_convert_type(jnp.int32(0x5F3759DF) - (i >> 1), jnp.float32)
    for _ in range(3):
        y = y * (1.5 - 0.5 * x * y * y)
    return jnp.where(x > 0.5, y, 0.0)


def _make_sc_kernel(NP, EP):
    R = NP // 16
    G2 = R // 16
    mesh = plsc.VectorSubcoreMesh(core_axis_name="c", subcore_axis_name="s")

    @functools.partial(
        pl.kernel,
        out_type=[
            jax.ShapeDtypeStruct((2, 2, NP), jnp.float32),
            jax.ShapeDtypeStruct((NP,), jnp.float32),
        ],
        mesh=mesh,
        scratch_types=[
            pltpu.VMEM((EP // 16,), jnp.int32),
            pltpu.VMEM((EP // 16,), jnp.int32),
            pltpu.VMEM((EP // 32,), jnp.int32),
            pltpu.VMEM((EP // 32,), jnp.int32),
            pltpu.VMEM((EP // 16,), jnp.float32),
            pltpu.VMEM((EP // 32,), jnp.float32),
            pltpu.VMEM((EP // 32,), jnp.float32),
            pltpu.VMEM((R,), jnp.float32),
            pltpu.VMEM((R,), jnp.float32),
            pltpu.VMEM((R,), jnp.float32),
            pltpu.VMEM((R,), jnp.float32),
            pltpu.VMEM((R,), jnp.float32),
            pltpu.VMEM((R,), jnp.float32),
            pltpu.VMEM((R,), jnp.float32),
            pltpu.VMEM_SHARED((NP,), jnp.float32),
            pltpu.VMEM_SHARED((NP,), jnp.float32),
            pltpu.VMEM_SHARED((NP,), jnp.float32),
            pltpu.VMEM_SHARED((NP,), jnp.float32),
            pltpu.VMEM_SHARED((NP,), jnp.float32),
            pltpu.VMEM_SHARED((NP,), jnp.float32),
        ],
    )
    def sc_kernel(u0_hbm, u1_hbm, src_hbm, dst_hbm, ones_hbm, zeros_hbm,
                  agg_out, nd_out,
                  sidx, didx, sidx3, didx3, onesv, msg0, msg1,
                  dov, div, u0v, u1v, us0v, us1v, ndv,
                  dego_s, degi_s, us0_s, us1_s, agg0_s, agg1_s):
        c = lax.axis_index("c")
        s = lax.axis_index("s")
        rows = pl.ds(s * R, R)

        pltpu.sync_copy(ones_hbm, onesv)
        pltpu.sync_copy(zeros_hbm.at[rows], dego_s.at[rows])
        pltpu.sync_copy(zeros_hbm.at[rows], degi_s.at[rows])
        pltpu.sync_copy(zeros_hbm.at[rows], agg0_s.at[rows])
        pltpu.sync_copy(zeros_hbm.at[rows], agg1_s.at[rows])
        plsc.subcore_barrier()

        B1 = EP // 16
        p1sl = pl.ds(s * B1, B1)
        pltpu.sync_copy(src_hbm.at[p1sl], sidx)
        pltpu.sync_copy(dst_hbm.at[p1sl], didx)
        pltpu.sync_copy(onesv, dego_s.at[sidx], add=True)
        pltpu.sync_copy(onesv, degi_s.at[didx], add=True)
        plsc.subcore_barrier()

        pltpu.sync_copy(dego_s.at[rows], dov)
        pltpu.sync_copy(degi_s.at[rows], div)
        pltpu.sync_copy(u0_hbm.at[rows], u0v)
        pltpu.sync_copy(u1_hbm.at[rows], u1v)

        def norm_g(g, carry):
            sl = pl.ds(g * 16, 16)
            nsrc = _rsqrt16(dov[sl])
            ndv[sl] = _rsqrt16(div[sl])
            us0v[sl] = u0v[sl] * nsrc
            us1v[sl] = u1v[sl] * nsrc
            return carry
        lax.fori_loop(0, G2, norm_g, 0)
        pltpu.sync_copy(us0v, us0_s.at[rows])
        pltpu.sync_copy(us1v, us1_s.at[rows])

        @pl.when(c == 0)
        def _():
            pltpu.sync_copy(ndv, nd_out.at[rows])
        plsc.subcore_barrier()

        wid = s * 2 + c
        B3 = EP // 32
        p3sl = pl.ds(wid * B3, B3)
        pltpu.sync_copy(src_hbm.at[p3sl], sidx3)
        pltpu.sync_copy(dst_hbm.at[p3sl], didx3)
        pltpu.sync_copy(us0_s.at[sidx3], msg0)
        pltpu.sync_copy(us1_s.at[sidx3], msg1)
        pltpu.sync_copy(msg0, agg0_s.at[didx3], add=True)
        pltpu.sync_copy(msg1, agg1_s.at[didx3], add=True)
        plsc.subcore_barrier()

        pltpu.sync_copy(agg0_s.at[rows], agg_out.at[c, 0, rows])
        pltpu.sync_copy(agg1_s.at[rows], agg_out.at[c, 1, rows])

    return sc_kernel


def kernel(x, edge_index, W_est, b_est, W_est_fc, b_est_fc, W_gnn, b_gnn,
           W_cls, b_cls):
    N, D = x.shape
    E = edge_index.shape[1]

    NP = 256 * ((N + 255) // 256)
    EP = 65536 * ((E + 65535) // 65536)
    if EP > E and NP == N:
        NP += 256

    xp = jnp.zeros((NP, D), x.dtype).at[:N].set(x)

    npad = EP - E
    if npad:
        pad = N + (jnp.arange(npad, dtype=jnp.int32) % (NP - N))
        src = jnp.concatenate([edge_index[0], pad])
        dst = jnp.concatenate([edge_index[1], pad])
    else:
        src = edge_index[0]
        dst = edge_index[1]

    u = pl.pallas_call(
        _u_body,
        out_shape=jax.ShapeDtypeStruct((2, NP), jnp.float32),
    )(xp, W_est, W_est_fc, W_gnn, W_cls)

    ones = jnp.ones((EP // 16,), jnp.float32)
    zeros = jnp.zeros((NP,), jnp.float32)
    agg, nd = _make_sc_kernel(NP, EP)(u[0], u[1], src, dst, ones, zeros)

    out2 = pl.pallas_call(
        _fin_body,
        out_shape=jax.ShapeDtypeStruct((2, NP), jnp.float32),
    )(agg, nd, b_est.reshape(1, -1), W_est_fc, b_est_fc.reshape(1, 1),
      b_gnn.reshape(1, -1), W_cls, b_cls.reshape(1, 1))

    return (out2[1, :N, None], out2[0, :N, None])
